# baseline (speedup 1.0000x reference)
import functools

import jax
import jax.numpy as jnp
from jax import lax
from jax.experimental import pallas as pl
from jax.experimental.pallas import tpu as pltpu
from jax.experimental.pallas import tpu_sc as plsc

N = 10000
E = 320000
NP = 10240
RPT = NP // 16
EP = 327680
EPT = EP // 16


def _make_prop_wide(dh, nb, cb, grp):
    nch = EPT // cb
    mesh = plsc.VectorSubcoreMesh(core_axis_name="c", subcore_axis_name="s")

    @functools.partial(
        pl.kernel,
        mesh=mesh,
        out_type=jax.ShapeDtypeStruct((2, NP, dh), jnp.float32),
        compiler_params=pltpu.CompilerParams(use_tc_tiling_on_sc=False),
        scratch_types=[
            pltpu.VMEM_SHARED((NP, dh), jnp.float32),
            pltpu.VMEM((grp, cb), jnp.int32),
            pltpu.VMEM((grp, cb), jnp.int32),
            pltpu.VMEM((nb, cb, dh), jnp.float32),
            pltpu.SemaphoreType.DMA((nb,)),
            pltpu.SemaphoreType.DMA((nb,)),
        ],
    )
    def k(tab, srcs, dsts, out, acc, src_m, dst_m, bufs, gsem, ssem):
        c = lax.axis_index("c")
        s = lax.axis_index("s")
        r0 = s * RPT
        pltpu.sync_copy(tab.at[pl.ds(c * NP + r0, RPT)], acc.at[pl.ds(r0, RPT)])
        plsc.subcore_barrier()

        def outer(g, carry):
            pltpu.sync_copy(srcs.at[c, s, pl.ds(g * grp, grp)], src_m)
            pltpu.sync_copy(dsts.at[s, pl.ds(g * grp, grp)], dst_m)
            ga = nb // 2
            gh, sh = {}, {}
            waited = set()
            for j in range(ga):
                gh[j] = pltpu.async_copy(tab.at[src_m.at[j]], bufs.at[j % nb],
                                         gsem.at[j % nb])
            for kk in range(grp):
                gh[kk].wait()
                sh[kk] = pltpu.async_copy(bufs.at[kk % nb],
                                          acc.at[dst_m.at[kk]],
                                          ssem.at[kk % nb], add=True)
                j = kk + ga
                if j < grp:
                    if j >= nb:
                        sh[j - nb].wait()
                        waited.add(j - nb)
                    gh[j] = pltpu.async_copy(tab.at[src_m.at[j]],
                                             bufs.at[j % nb], gsem.at[j % nb])
            for kk in range(grp):
                if kk not in waited:
                    sh[kk].wait()
            return carry

        lax.fori_loop(0, nch // grp, outer, 0)
        plsc.subcore_barrier()
        pltpu.sync_copy(acc.at[pl.ds(r0, RPT)], out.at[c, pl.ds(r0, RPT)])

    return k


_prop32 = _make_prop_wide(32, 6, 128, 16)
_prop128 = _make_prop_wide(128, 4, 64, 32)


def _make_prop_d1():
    mesh = plsc.VectorSubcoreMesh(core_axis_name="c", subcore_axis_name="s")
    nstep = EP // 32 // 16

    @functools.partial(
        pl.kernel,
        mesh=mesh,
        out_type=jax.ShapeDtypeStruct((2, NP), jnp.float32),
        compiler_params=pltpu.CompilerParams(use_tc_tiling_on_sc=False,
                                             needs_layout_passes=False),
        scratch_types=[
            pltpu.VMEM_SHARED((16, NP), jnp.float32),
            pltpu.VMEM((NP,), jnp.float32),
            pltpu.VMEM((NP,), jnp.float32),
            pltpu.VMEM((nstep, 16), jnp.int32),
            pltpu.VMEM((nstep, 16), jnp.int32),
            pltpu.VMEM((16, RPT), jnp.float32),
            pltpu.VMEM((RPT,), jnp.float32),
        ],
    )
    def k(tab, srcs, dsts, out, part, tabv, loc, src_m, dst_m, tmp, res):
        c = lax.axis_index("c")
        s = lax.axis_index("s")
        w = 2 * s + c
        pltpu.sync_copy(tab, tabv)
        pltpu.sync_copy(srcs.at[w], src_m)
        pltpu.sync_copy(dsts.at[w], dst_m)

        def z(j, carry):
            loc[pl.ds(j * 16, 16)] = jnp.zeros((16,), jnp.float32)
            return carry

        lax.fori_loop(0, NP // 16, z, 0)

        def step(j, carry):
            g = plsc.load_gather(tabv, [src_m[j]])
            plsc.addupdate_scatter(loc, [dst_m[j]], g)
            return carry

        lax.fori_loop(0, nstep, step, 0)
        pltpu.sync_copy(loc, part.at[s])
        plsc.subcore_barrier()

        r0 = s * RPT
        for t in range(16):
            pltpu.sync_copy(part.at[t, pl.ds(r0, RPT)], tmp.at[t])

        def red(j, carry):
            v = tmp[0, pl.ds(j * 16, 16)]
            for t in range(1, 16):
                v = v + tmp[t, pl.ds(j * 16, 16)]
            res[pl.ds(j * 16, 16)] = v
            return carry

        lax.fori_loop(0, RPT // 16, red, 0)
        pltpu.sync_copy(res, out.at[c, pl.ds(r0, RPT)])

    return k


_prop_d1 = _make_prop_d1()


def _pad_tab(tab_ref, u, dh):
    tab_ref[pl.ds(0, N)] = u[:, :dh]
    tab_ref[pl.ds(NP, N)] = u[:, dh:]
    zpad = jnp.zeros((NP - N, dh), jnp.float32)
    tab_ref[pl.ds(N, NP - N)] = zpad
    tab_ref[pl.ds(NP + N, NP - N)] = zpad


def _merge(s_ref):
    v = s_ref[...]
    return jnp.concatenate([v[0, :N], v[1, :N]], axis=1)


def _t1(x_ref, w_ref, cnt_ref, dis_ref, tab_ref):
    deg = 1.0 + cnt_ref[0] + cnt_ref[1]
    dis = lax.rsqrt(deg)
    dis_ref[...] = dis
    z = jnp.dot(x_ref[...], w_ref[...], preferred_element_type=jnp.float32)
    u = z * dis[:N]
    _pad_tab(tab_ref, u, 32)


def _t2(s_ref, dis_ref, b_ref, w_ref, tab_ref):
    dis = dis_ref[...]
    h = jnp.maximum(dis[:N] * _merge(s_ref) + b_ref[...], 0.0)
    u = dis[:N] * jnp.dot(h, w_ref[...], preferred_element_type=jnp.float32)
    _pad_tab(tab_ref, u, 32)


def _t3(s_ref, dis_ref, b_ref, tab_ref):
    dis = dis_ref[...]
    p = dis[:N] * _merge(s_ref) + b_ref[...]
    h = jnp.where(p > 0, p, 0.1 * p)
    _pad_tab(tab_ref, dis[:N] * h, 32)


def _t4(s_ref, dis_ref, ba_ref, wa_ref, wb_ref, tab_ref):
    dis = dis_ref[...]
    p = dis[:N] * _merge(s_ref)
    h = jnp.maximum(jnp.dot(p, wa_ref[...], preferred_element_type=jnp.float32)
                    + ba_ref[...], 0.0)
    u = dis[:N] * jnp.dot(h, wb_ref[...], preferred_element_type=jnp.float32)
    _pad_tab(tab_ref, u, 128)


def _t5(s_ref, dis_ref, b_ref, tab_ref):
    dis = dis_ref[...]
    p = dis[:N] * _merge(s_ref) + b_ref[...]
    h = jnp.where(p > 0, p, 0.1 * p)
    _pad_tab(tab_ref, dis[:N] * h, 128)


def _t6(s_ref, dis_ref, ba_ref, wa_ref, wb_ref, tab_ref):
    dis = dis_ref[...]
    p = dis[:N] * _merge(s_ref)
    h = jnp.maximum(jnp.dot(p, wa_ref[...], preferred_element_type=jnp.float32)
                    + ba_ref[...], 0.0)
    z = jnp.sum(h * wb_ref[...], axis=1, keepdims=True)
    tab_ref[pl.ds(0, N)] = dis[:N] * z
    tab_ref[pl.ds(N, NP - N)] = jnp.zeros((NP - N, 1), jnp.float32)


def _t7(t6_ref, u6_ref, dis_ref, b_ref, out_ref):
    v = t6_ref[...]
    out_ref[...] = (dis_ref[pl.ds(0, N)]
                    * (u6_ref[pl.ds(0, N)] + v[0, :N] + v[1, :N]) + b_ref[...])


def _tc(body, out_shape, *args):
    return pl.pallas_call(body, out_shape=out_shape)(*args)


def kernel(x, edge_index, W0a, b0a, W0b, b0b, W1a, b1a, W1b, b1b, W2a, b2a, W2b, b2b):
    f32 = jnp.float32
    src = edge_index[0]
    dst = edge_index[1]
    srcp = jnp.concatenate([src, jnp.full((EP - E,), N, jnp.int32)])
    dstp = jnp.concatenate([dst, jnp.full((EP - E,), N, jnp.int32)])
    offs = jnp.array([0, NP], jnp.int32)
    srcs_sh = srcp[None, :] + offs[:, None]
    srcs_w32 = srcs_sh.reshape(2, 16, EPT // 128, 128)
    dst_w32 = dstp.reshape(16, EPT // 128, 128)
    srcs_w128 = srcs_sh.reshape(2, 16, EPT // 64, 64)
    dst_w128 = dstp.reshape(16, EPT // 64, 64)
    src_1 = srcp.reshape(32, EP // 32 // 16, 16)
    dst_1 = dstp.reshape(32, EP // 32 // 16, 16)

    b0a_ = b0a.reshape(1, -1)
    b0b_ = b0b.reshape(1, -1)
    b1a_ = b1a.reshape(1, -1)
    b1b_ = b1b.reshape(1, -1)
    b2a_ = b2a.reshape(1, -1)
    b2b_ = b2b.reshape(1, -1)
    w2b_row = W2b.reshape(1, -1)

    ones_tab = jnp.zeros((NP,), f32).at[:N].set(1.0)
    cnt = _prop_d1(ones_tab, src_1, dst_1)
    cnt3 = cnt.reshape(2, NP, 1)

    sd = jax.ShapeDtypeStruct
    dis, tab1 = _tc(_t1, [sd((NP, 1), f32), sd((2 * NP, 32), f32)],
                    x, W0a, cnt3)
    s1 = _prop32(tab1, srcs_w32, dst_w32)
    tab2 = _tc(_t2, sd((2 * NP, 32), f32), s1, dis, b0a_, W0b)
    s2 = _prop32(tab2, srcs_w32, dst_w32)
    tab3 = _tc(_t3, sd((2 * NP, 32), f32), s2, dis, b0b_)
    s3 = _prop32(tab3, srcs_w32, dst_w32)
    tab4 = _tc(_t4, sd((2 * NP, 128), f32), s3, dis, b1a_, W1a, W1b)
    s4 = _prop128(tab4, srcs_w128, dst_w128)
    tab5 = _tc(_t5, sd((2 * NP, 128), f32), s4, dis, b1b_)
    s5 = _prop128(tab5, srcs_w128, dst_w128)
    tab6 = _tc(_t6, sd((NP, 1), f32), s5, dis, b2a_, W2a, w2b_row)
    t6 = _prop_d1(tab6.reshape(NP), src_1, dst_1)
    out = _tc(_t7, sd((N, 1), f32), t6.reshape(2, NP, 1), tab6, dis, b2b_)
    return out


# speedup vs baseline: 1.2179x; 1.2179x over previous
import functools

import jax
import jax.numpy as jnp
from jax import lax
from jax.experimental import pallas as pl
from jax.experimental.pallas import tpu as pltpu
from jax.experimental.pallas import tpu_sc as plsc

N = 10000
E = 320000
NP = 10240
RPT = NP // 16
EP = 327680
EPT = EP // 16


def _make_prop_wide(dh, nb, cb, grp, spmem_tab):
    nch = EPT // cb
    mesh = plsc.VectorSubcoreMesh(core_axis_name="c", subcore_axis_name="s")
    scratch = [
        pltpu.VMEM_SHARED((NP, dh), jnp.float32),
        pltpu.VMEM((grp, cb), jnp.int32),
        pltpu.VMEM((grp, cb), jnp.int32),
        pltpu.VMEM((nb, cb, dh), jnp.float32),
        pltpu.SemaphoreType.DMA((nb,)),
        pltpu.SemaphoreType.DMA((nb,)),
    ]
    if spmem_tab:
        scratch.append(pltpu.VMEM_SHARED((NP, dh), jnp.float32))

    @functools.partial(
        pl.kernel,
        mesh=mesh,
        out_type=jax.ShapeDtypeStruct((2, NP, dh), jnp.float32),
        compiler_params=pltpu.CompilerParams(use_tc_tiling_on_sc=False),
        scratch_types=scratch,
    )
    def k(tab, srcs, dsts, out, acc, src_m, dst_m, bufs, gsem, ssem,
          *maybe_tab_sh):
        c = lax.axis_index("c")
        s = lax.axis_index("s")
        r0 = s * RPT
        pltpu.sync_copy(tab.at[pl.ds(c * NP + r0, RPT)], acc.at[pl.ds(r0, RPT)])
        if spmem_tab:
            gsrc = maybe_tab_sh[0]
            pltpu.sync_copy(tab.at[pl.ds(c * NP + r0, RPT)],
                            gsrc.at[pl.ds(r0, RPT)])
        else:
            gsrc = tab
        plsc.subcore_barrier()

        def outer(g, carry):
            if spmem_tab:
                pltpu.sync_copy(srcs.at[s, pl.ds(g * grp, grp)], src_m)
            else:
                pltpu.sync_copy(srcs.at[c, s, pl.ds(g * grp, grp)], src_m)
            pltpu.sync_copy(dsts.at[s, pl.ds(g * grp, grp)], dst_m)
            ga = nb // 2
            gh, sh = {}, {}
            waited = set()
            for j in range(ga):
                gh[j] = pltpu.async_copy(gsrc.at[src_m.at[j]], bufs.at[j % nb],
                                         gsem.at[j % nb])
            for kk in range(grp):
                gh[kk].wait()
                sh[kk] = pltpu.async_copy(bufs.at[kk % nb],
                                          acc.at[dst_m.at[kk]],
                                          ssem.at[kk % nb], add=True)
                j = kk + ga
                if j < grp:
                    if j >= nb:
                        sh[j - nb].wait()
                        waited.add(j - nb)
                    gh[j] = pltpu.async_copy(gsrc.at[src_m.at[j]],
                                             bufs.at[j % nb], gsem.at[j % nb])
            for kk in range(grp):
                if kk not in waited:
                    sh[kk].wait()
            return carry

        lax.fori_loop(0, nch // grp, outer, 0)
        plsc.subcore_barrier()
        pltpu.sync_copy(acc.at[pl.ds(r0, RPT)], out.at[c, pl.ds(r0, RPT)])

    return k


_prop32 = _make_prop_wide(32, 6, 128, 16, True)
_prop128 = _make_prop_wide(128, 4, 64, 32, False)


def _make_prop_d1():
    mesh = plsc.VectorSubcoreMesh(core_axis_name="c", subcore_axis_name="s")
    nstep = EP // 32 // 16

    @functools.partial(
        pl.kernel,
        mesh=mesh,
        out_type=jax.ShapeDtypeStruct((2, NP), jnp.float32),
        compiler_params=pltpu.CompilerParams(use_tc_tiling_on_sc=False,
                                             needs_layout_passes=False),
        scratch_types=[
            pltpu.VMEM_SHARED((16, NP), jnp.float32),
            pltpu.VMEM((NP,), jnp.float32),
            pltpu.VMEM((NP,), jnp.float32),
            pltpu.VMEM((nstep, 16), jnp.int32),
            pltpu.VMEM((nstep, 16), jnp.int32),
            pltpu.VMEM((16, RPT), jnp.float32),
            pltpu.VMEM((RPT,), jnp.float32),
        ],
    )
    def k(tab, srcs, dsts, out, part, tabv, loc, src_m, dst_m, tmp, res):
        c = lax.axis_index("c")
        s = lax.axis_index("s")
        w = 2 * s + c
        pltpu.sync_copy(tab, tabv)
        pltpu.sync_copy(srcs.at[w], src_m)
        pltpu.sync_copy(dsts.at[w], dst_m)

        def z(j, carry):
            loc[pl.ds(j * 16, 16)] = jnp.zeros((16,), jnp.float32)
            return carry

        lax.fori_loop(0, NP // 16, z, 0)

        def step(j, carry):
            g = plsc.load_gather(tabv, [src_m[j]])
            plsc.addupdate_scatter(loc, [dst_m[j]], g)
            return carry

        lax.fori_loop(0, nstep, step, 0)
        pltpu.sync_copy(loc, part.at[s])
        plsc.subcore_barrier()

        r0 = s * RPT
        for t in range(16):
            pltpu.sync_copy(part.at[t, pl.ds(r0, RPT)], tmp.at[t])

        def red(j, carry):
            v = tmp[0, pl.ds(j * 16, 16)]
            for t in range(1, 16):
                v = v + tmp[t, pl.ds(j * 16, 16)]
            res[pl.ds(j * 16, 16)] = v
            return carry

        lax.fori_loop(0, RPT // 16, red, 0)
        pltpu.sync_copy(res, out.at[c, pl.ds(r0, RPT)])

    return k


_prop_d1 = _make_prop_d1()


def _pad_tab(tab_ref, u, dh):
    tab_ref[pl.ds(0, N)] = u[:, :dh]
    tab_ref[pl.ds(NP, N)] = u[:, dh:]
    zpad = jnp.zeros((NP - N, dh), jnp.float32)
    tab_ref[pl.ds(N, NP - N)] = zpad
    tab_ref[pl.ds(NP + N, NP - N)] = zpad


def _merge(s_ref):
    v = s_ref[...]
    return jnp.concatenate([v[0, :N], v[1, :N]], axis=1)


def _t1(x_ref, w_ref, cnt_ref, dis_ref, tab_ref):
    deg = 1.0 + cnt_ref[0] + cnt_ref[1]
    dis = lax.rsqrt(deg)
    dis_ref[...] = dis
    z = jnp.dot(x_ref[...], w_ref[...], preferred_element_type=jnp.float32)
    u = z * dis[:N]
    _pad_tab(tab_ref, u, 32)


def _t2(s_ref, dis_ref, b_ref, w_ref, tab_ref):
    dis = dis_ref[...]
    h = jnp.maximum(dis[:N] * _merge(s_ref) + b_ref[...], 0.0)
    u = dis[:N] * jnp.dot(h, w_ref[...], preferred_element_type=jnp.float32)
    _pad_tab(tab_ref, u, 32)


def _t3(s_ref, dis_ref, b_ref, tab_ref):
    dis = dis_ref[...]
    p = dis[:N] * _merge(s_ref) + b_ref[...]
    h = jnp.where(p > 0, p, 0.1 * p)
    _pad_tab(tab_ref, dis[:N] * h, 32)


def _t4(s_ref, dis_ref, ba_ref, wa_ref, wb_ref, tab_ref):
    dis = dis_ref[...]
    p = dis[:N] * _merge(s_ref)
    h = jnp.maximum(jnp.dot(p, wa_ref[...], preferred_element_type=jnp.float32)
                    + ba_ref[...], 0.0)
    u = dis[:N] * jnp.dot(h, wb_ref[...], preferred_element_type=jnp.float32)
    _pad_tab(tab_ref, u, 128)


def _t5(s_ref, dis_ref, b_ref, tab_ref):
    dis = dis_ref[...]
    p = dis[:N] * _merge(s_ref) + b_ref[...]
    h = jnp.where(p > 0, p, 0.1 * p)
    _pad_tab(tab_ref, dis[:N] * h, 128)


def _t6(s_ref, dis_ref, ba_ref, wa_ref, wb_ref, tab_ref):
    dis = dis_ref[...]
    p = dis[:N] * _merge(s_ref)
    h = jnp.maximum(jnp.dot(p, wa_ref[...], preferred_element_type=jnp.float32)
                    + ba_ref[...], 0.0)
    z = jnp.sum(h * wb_ref[...], axis=1, keepdims=True)
    tab_ref[pl.ds(0, N)] = dis[:N] * z
    tab_ref[pl.ds(N, NP - N)] = jnp.zeros((NP - N, 1), jnp.float32)


def _t7(t6_ref, u6_ref, dis_ref, b_ref, out_ref):
    v = t6_ref[...]
    out_ref[...] = (dis_ref[pl.ds(0, N)]
                    * (u6_ref[pl.ds(0, N)] + v[0, :N] + v[1, :N]) + b_ref[...])


def _tc(body, out_shape, *args):
    return pl.pallas_call(body, out_shape=out_shape)(*args)


def kernel(x, edge_index, W0a, b0a, W0b, b0b, W1a, b1a, W1b, b1b, W2a, b2a, W2b, b2b):
    f32 = jnp.float32
    src = edge_index[0]
    dst = edge_index[1]
    srcp = jnp.concatenate([src, jnp.full((EP - E,), N, jnp.int32)])
    dstp = jnp.concatenate([dst, jnp.full((EP - E,), N, jnp.int32)])
    offs = jnp.array([0, NP], jnp.int32)
    srcs_sh = srcp[None, :] + offs[:, None]
    dst_w32_src = srcp.reshape(16, EPT // 128, 128)
    dst_w32 = dstp.reshape(16, EPT // 128, 128)
    srcs_w128 = srcs_sh.reshape(2, 16, EPT // 64, 64)
    dst_w128 = dstp.reshape(16, EPT // 64, 64)
    src_1 = srcp.reshape(32, EP // 32 // 16, 16)
    dst_1 = dstp.reshape(32, EP // 32 // 16, 16)

    b0a_ = b0a.reshape(1, -1)
    b0b_ = b0b.reshape(1, -1)
    b1a_ = b1a.reshape(1, -1)
    b1b_ = b1b.reshape(1, -1)
    b2a_ = b2a.reshape(1, -1)
    b2b_ = b2b.reshape(1, -1)
    w2b_row = W2b.reshape(1, -1)

    ones_tab = jnp.zeros((NP,), f32).at[:N].set(1.0)
    cnt = _prop_d1(ones_tab, src_1, dst_1)
    cnt3 = cnt.reshape(2, NP, 1)

    sd = jax.ShapeDtypeStruct
    dis, tab1 = _tc(_t1, [sd((NP, 1), f32), sd((2 * NP, 32), f32)],
                    x, W0a, cnt3)
    s1 = _prop32(tab1, dst_w32_src, dst_w32)
    tab2 = _tc(_t2, sd((2 * NP, 32), f32), s1, dis, b0a_, W0b)
    s2 = _prop32(tab2, dst_w32_src, dst_w32)
    tab3 = _tc(_t3, sd((2 * NP, 32), f32), s2, dis, b0b_)
    s3 = _prop32(tab3, dst_w32_src, dst_w32)
    tab4 = _tc(_t4, sd((2 * NP, 128), f32), s3, dis, b1a_, W1a, W1b)
    s4 = _prop128(tab4, srcs_w128, dst_w128)
    tab5 = _tc(_t5, sd((2 * NP, 128), f32), s4, dis, b1b_)
    s5 = _prop128(tab5, srcs_w128, dst_w128)
    tab6 = _tc(_t6, sd((NP, 1), f32), s5, dis, b2a_, W2a, w2b_row)
    t6 = _prop_d1(tab6.reshape(NP), src_1, dst_1)
    out = _tc(_t7, sd((N, 1), f32), t6.reshape(2, NP, 1), tab6, dis, b2b_)
    return out


# speedup vs baseline: 1.9335x; 1.5876x over previous
import functools

import jax
import jax.numpy as jnp
from jax import lax
from jax.experimental import pallas as pl
from jax.experimental.pallas import tpu as pltpu
from jax.experimental.pallas import tpu_sc as plsc

N = 10000
E = 320000
NP = 10240
RPT = NP // 16
EP = 327680
EPT = EP // 16


def _make_prop_wide(dh, nb, cb, grp, nplanes, pb):
    nch = EPT // cb
    mesh = plsc.VectorSubcoreMesh(core_axis_name="c", subcore_axis_name="s")
    scratch = [
        pltpu.VMEM_SHARED((NP, dh), jnp.float32),
        pltpu.VMEM((grp, cb), jnp.int32),
        pltpu.VMEM((grp, cb), jnp.int32),
        pltpu.VMEM((nb, cb, dh), jnp.float32),
        pltpu.SemaphoreType.DMA((nb,)),
        pltpu.SemaphoreType.DMA((nb,)),
        pltpu.VMEM_SHARED((NP, dh), jnp.float32),
    ]

    @functools.partial(
        pl.kernel,
        mesh=mesh,
        out_type=jax.ShapeDtypeStruct((2, NP, dh), jnp.float32),
        compiler_params=pltpu.CompilerParams(use_tc_tiling_on_sc=False),
        scratch_types=scratch,
    )
    def k(tab, srcs, dsts, out, acc, src_m, dst_m, bufs, gsem, ssem, gsrc):
        c = lax.axis_index("c")
        s = lax.axis_index("s")
        r0 = s * RPT
        p0 = (pb + c) * NP
        pltpu.sync_copy(tab.at[pl.ds(p0 + r0, RPT)], acc.at[pl.ds(r0, RPT)])
        pltpu.sync_copy(tab.at[pl.ds(p0 + r0, RPT)], gsrc.at[pl.ds(r0, RPT)])
        plsc.subcore_barrier()

        def outer(g, carry):
            pltpu.sync_copy(srcs.at[s, pl.ds(g * grp, grp)], src_m)
            pltpu.sync_copy(dsts.at[s, pl.ds(g * grp, grp)], dst_m)
            ga = nb // 2
            gh, sh = {}, {}
            waited = set()
            for j in range(ga):
                gh[j] = pltpu.async_copy(gsrc.at[src_m.at[j]], bufs.at[j % nb],
                                         gsem.at[j % nb])
            for kk in range(grp):
                gh[kk].wait()
                sh[kk] = pltpu.async_copy(bufs.at[kk % nb],
                                          acc.at[dst_m.at[kk]],
                                          ssem.at[kk % nb], add=True)
                j = kk + ga
                if j < grp:
                    if j >= nb:
                        sh[j - nb].wait()
                        waited.add(j - nb)
                    gh[j] = pltpu.async_copy(gsrc.at[src_m.at[j]],
                                             bufs.at[j % nb], gsem.at[j % nb])
            for kk in range(grp):
                if kk not in waited:
                    sh[kk].wait()
            return carry

        lax.fori_loop(0, nch // grp, outer, 0)
        plsc.subcore_barrier()
        pltpu.sync_copy(acc.at[pl.ds(r0, RPT)], out.at[c, pl.ds(r0, RPT)])

    return k


_prop32 = _make_prop_wide(32, 6, 128, 16, 2, 0)
_prop64a = _make_prop_wide(64, 4, 128, 16, 4, 0)
_prop64b = _make_prop_wide(64, 4, 128, 16, 4, 2)


def _make_prop_d1():
    mesh = plsc.VectorSubcoreMesh(core_axis_name="c", subcore_axis_name="s")
    nstep = EP // 32 // 16

    @functools.partial(
        pl.kernel,
        mesh=mesh,
        out_type=jax.ShapeDtypeStruct((2, NP), jnp.float32),
        compiler_params=pltpu.CompilerParams(use_tc_tiling_on_sc=False,
                                             needs_layout_passes=False),
        scratch_types=[
            pltpu.VMEM_SHARED((16, NP), jnp.float32),
            pltpu.VMEM((NP,), jnp.float32),
            pltpu.VMEM((NP,), jnp.float32),
            pltpu.VMEM((nstep, 16), jnp.int32),
            pltpu.VMEM((nstep, 16), jnp.int32),
            pltpu.VMEM((16, RPT), jnp.float32),
            pltpu.VMEM((RPT,), jnp.float32),
        ],
    )
    def k(tab, srcs, dsts, out, part, tabv, loc, src_m, dst_m, tmp, res):
        c = lax.axis_index("c")
        s = lax.axis_index("s")
        w = 2 * s + c
        pltpu.sync_copy(tab, tabv)
        pltpu.sync_copy(srcs.at[w], src_m)
        pltpu.sync_copy(dsts.at[w], dst_m)

        def z(j, carry):
            loc[pl.ds(j * 16, 16)] = jnp.zeros((16,), jnp.float32)
            return carry

        lax.fori_loop(0, NP // 16, z, 0)

        def step(j, carry):
            g = plsc.load_gather(tabv, [src_m[j]])
            plsc.addupdate_scatter(loc, [dst_m[j]], g)
            return carry

        lax.fori_loop(0, nstep, step, 0)
        pltpu.sync_copy(loc, part.at[s])
        plsc.subcore_barrier()

        r0 = s * RPT
        for t in range(16):
            pltpu.sync_copy(part.at[t, pl.ds(r0, RPT)], tmp.at[t])

        def red(j, carry):
            v = tmp[0, pl.ds(j * 16, 16)]
            for t in range(1, 16):
                v = v + tmp[t, pl.ds(j * 16, 16)]
            res[pl.ds(j * 16, 16)] = v
            return carry

        lax.fori_loop(0, RPT // 16, red, 0)
        pltpu.sync_copy(res, out.at[c, pl.ds(r0, RPT)])

    return k


_prop_d1 = _make_prop_d1()


def _pad_tab(tab_ref, u, dh, nplanes=2):
    zpad = jnp.zeros((NP - N, dh), jnp.float32)
    for q in range(nplanes):
        tab_ref[pl.ds(q * NP, N)] = u[:, q * dh:(q + 1) * dh]
        tab_ref[pl.ds(q * NP + N, NP - N)] = zpad


def _merge(s_ref):
    v = s_ref[...]
    return jnp.concatenate([v[0, :N], v[1, :N]], axis=1)


def _merge4(a_ref, b_ref):
    va = a_ref[...]
    vb = b_ref[...]
    return jnp.concatenate([va[0, :N], va[1, :N], vb[0, :N], vb[1, :N]],
                           axis=1)


def _t1(x_ref, w_ref, cnt_ref, dis_ref, tab_ref):
    deg = 1.0 + cnt_ref[0] + cnt_ref[1]
    dis = lax.rsqrt(deg)
    dis_ref[...] = dis
    z = jnp.dot(x_ref[...], w_ref[...], preferred_element_type=jnp.float32)
    u = z * dis[:N]
    _pad_tab(tab_ref, u, 32)


def _t2(s_ref, dis_ref, b_ref, w_ref, tab_ref):
    dis = dis_ref[...]
    h = jnp.maximum(dis[:N] * _merge(s_ref) + b_ref[...], 0.0)
    u = dis[:N] * jnp.dot(h, w_ref[...], preferred_element_type=jnp.float32)
    _pad_tab(tab_ref, u, 32)


def _t3(s_ref, dis_ref, b_ref, tab_ref):
    dis = dis_ref[...]
    p = dis[:N] * _merge(s_ref) + b_ref[...]
    h = jnp.where(p > 0, p, 0.1 * p)
    _pad_tab(tab_ref, dis[:N] * h, 32)


def _t4(s_ref, dis_ref, ba_ref, wa_ref, wb_ref, tab_ref):
    dis = dis_ref[...]
    p = dis[:N] * _merge(s_ref)
    h = jnp.maximum(jnp.dot(p, wa_ref[...], preferred_element_type=jnp.float32)
                    + ba_ref[...], 0.0)
    u = dis[:N] * jnp.dot(h, wb_ref[...], preferred_element_type=jnp.float32)
    _pad_tab(tab_ref, u, 64, 4)


def _t5(sa_ref, sb_ref, dis_ref, b_ref, tab_ref):
    dis = dis_ref[...]
    p = dis[:N] * _merge4(sa_ref, sb_ref) + b_ref[...]
    h = jnp.where(p > 0, p, 0.1 * p)
    _pad_tab(tab_ref, dis[:N] * h, 64, 4)


def _t6(sa_ref, sb_ref, dis_ref, ba_ref, wa_ref, wb_ref, tab_ref):
    dis = dis_ref[...]
    p = dis[:N] * _merge4(sa_ref, sb_ref)
    h = jnp.maximum(jnp.dot(p, wa_ref[...], preferred_element_type=jnp.float32)
                    + ba_ref[...], 0.0)
    z = jnp.sum(h * wb_ref[...], axis=1, keepdims=True)
    tab_ref[pl.ds(0, N)] = dis[:N] * z
    tab_ref[pl.ds(N, NP - N)] = jnp.zeros((NP - N, 1), jnp.float32)


def _t7(t6_ref, u6_ref, dis_ref, b_ref, out_ref):
    v = t6_ref[...]
    out_ref[...] = (dis_ref[pl.ds(0, N)]
                    * (u6_ref[pl.ds(0, N)] + v[0, :N] + v[1, :N]) + b_ref[...])


def _tc(body, out_shape, *args):
    return pl.pallas_call(
        body, out_shape=out_shape,
        compiler_params=pltpu.CompilerParams(
            vmem_limit_bytes=100 * 1024 * 1024),
    )(*args)


def kernel(x, edge_index, W0a, b0a, W0b, b0b, W1a, b1a, W1b, b1b, W2a, b2a, W2b, b2b):
    f32 = jnp.float32
    src = edge_index[0]
    dst = edge_index[1]
    srcp = jnp.concatenate([src, jnp.full((EP - E,), N, jnp.int32)])
    dstp = jnp.concatenate([dst, jnp.full((EP - E,), N, jnp.int32)])
    offs = jnp.array([0, NP], jnp.int32)
    src_w = srcp.reshape(16, EPT // 128, 128)
    dst_w = dstp.reshape(16, EPT // 128, 128)
    src_1 = srcp.reshape(32, EP // 32 // 16, 16)
    dst_1 = dstp.reshape(32, EP // 32 // 16, 16)

    b0a_ = b0a.reshape(1, -1)
    b0b_ = b0b.reshape(1, -1)
    b1a_ = b1a.reshape(1, -1)
    b1b_ = b1b.reshape(1, -1)
    b2a_ = b2a.reshape(1, -1)
    b2b_ = b2b.reshape(1, -1)
    w2b_row = W2b.reshape(1, -1)

    ones_tab = jnp.zeros((NP,), f32).at[:N].set(1.0)
    cnt = _prop_d1(ones_tab, src_1, dst_1)
    cnt3 = cnt.reshape(2, NP, 1)

    sd = jax.ShapeDtypeStruct
    dis, tab1 = _tc(_t1, [sd((NP, 1), f32), sd((2 * NP, 32), f32)],
                    x, W0a, cnt3)
    s1 = _prop32(tab1, src_w, dst_w)
    tab2 = _tc(_t2, sd((2 * NP, 32), f32), s1, dis, b0a_, W0b)
    s2 = _prop32(tab2, src_w, dst_w)
    tab3 = _tc(_t3, sd((2 * NP, 32), f32), s2, dis, b0b_)
    s3 = _prop32(tab3, src_w, dst_w)
    tab4 = _tc(_t4, sd((4 * NP, 64), f32), s3, dis, b1a_, W1a, W1b)
    s4a = _prop64a(tab4, src_w, dst_w)
    s4b = _prop64b(tab4, src_w, dst_w)
    tab5 = _tc(_t5, sd((4 * NP, 64), f32), s4a, s4b, dis, b1b_)
    s5a = _prop64a(tab5, src_w, dst_w)
    s5b = _prop64b(tab5, src_w, dst_w)
    tab6 = _tc(_t6, sd((NP, 1), f32), s5a, s5b, dis, b2a_, W2a, w2b_row)
    t6 = _prop_d1(tab6.reshape(NP), src_1, dst_1)
    out = _tc(_t7, sd((N, 1), f32), t6.reshape(2, NP, 1), tab6, dis, b2b_)
    return out
